# baseline (speedup 1.0000x reference)
import functools

import jax
import jax.numpy as jnp
from jax import lax
from jax.experimental import pallas as pl
from jax.experimental.pallas import tpu as pltpu
from jax.experimental.pallas import tpu_sc as plsc

N = 2048
DIM = 768
E = 8
FF = DIM * 4
THRESHOLD = 0.3

NC = 2
NS = 16
L = 16
NW = NC * NS
CHW = N // NW
NG = CHW // L

BTG = 256
NIG = N // BTG

BTC = 256
NJ = N // BTC
FFB = 1536
NK = FF // FFB


def _gate_kernel(x_ref, wg_ref, bg_ref, scores_ref, w_ref, pos_ref, cnt_ref,
                 carry_ref):
    i = pl.program_id(0)

    @pl.when(i == 0)
    def _init():
        carry_ref[...] = jnp.zeros((E, 1), jnp.float32)

    logits = lax.dot_general(wg_ref[...], x_ref[...],
                             (((0,), (1,)), ((), ())),
                             preferred_element_type=jnp.float32)
    logits = logits + bg_ref[...]
    mx0 = jnp.max(logits, axis=0, keepdims=True)
    ex = jnp.exp(logits - mx0)
    scores = ex / jnp.sum(ex, axis=0, keepdims=True)
    scores_ref[...] = scores

    mask = (scores >= THRESHOLD).astype(jnp.float32)
    masked = scores * mask
    denom_raw = jnp.sum(masked, axis=0, keepdims=True)
    normed = masked / (denom_raw + 1e-6)
    iot = lax.broadcasted_iota(jnp.int32, scores.shape, 0)
    mxs = jnp.max(scores, axis=0, keepdims=True)
    cand = jnp.where(scores == mxs, iot, E)
    top1 = jnp.min(cand, axis=0, keepdims=True)
    onehot = (iot == top1).astype(jnp.float32)
    w = jnp.where(denom_raw == 0.0, onehot, normed)
    w_ref[...] = w

    act = (w > 0.0).astype(jnp.float32)
    rowi = lax.broadcasted_iota(jnp.int32, (BTG, BTG), 0)
    coli = lax.broadcasted_iota(jnp.int32, (BTG, BTG), 1)
    tri = (rowi < coli).astype(jnp.float32)
    pos = lax.dot_general(act, tri, (((1,), (0,)), ((), ())),
                          preferred_element_type=jnp.float32)
    pos = pos + carry_ref[...]
    pos_ref[...] = pos.astype(jnp.int32)
    new_carry = carry_ref[...] + jnp.sum(act, axis=1, keepdims=True)
    carry_ref[...] = new_carry

    @pl.when(i == NIG - 1)
    def _fin():
        cnt_ref[...] = new_carry.astype(jnp.int32)


def _gate(x, Wg, bg):
    return pl.pallas_call(
        _gate_kernel,
        grid=(NIG,),
        in_specs=[
            pl.BlockSpec((BTG, DIM), lambda i: (i, 0)),
            pl.BlockSpec((DIM, E), lambda i: (0, 0)),
            pl.BlockSpec((E, 1), lambda i: (0, 0)),
        ],
        out_specs=(
            pl.BlockSpec((E, BTG), lambda i: (0, i)),
            pl.BlockSpec((E, BTG), lambda i: (0, i)),
            pl.BlockSpec((E, BTG), lambda i: (0, i)),
            pl.BlockSpec((E, 1), lambda i: (0, 0)),
        ),
        out_shape=(
            jax.ShapeDtypeStruct((E, N), jnp.float32),
            jax.ShapeDtypeStruct((E, N), jnp.float32),
            jax.ShapeDtypeStruct((E, N), jnp.int32),
            jax.ShapeDtypeStruct((E, 1), jnp.int32),
        ),
        scratch_shapes=[pltpu.VMEM((E, 1), jnp.float32)],
    )(x, Wg, bg.reshape(E, 1))


_SC_MESH = plsc.VectorSubcoreMesh(core_axis_name="c", subcore_axis_name="s")
_SC_PARAMS = pltpu.CompilerParams(needs_layout_passes=False)
IGN = 2 ** 30


@functools.partial(
    pl.kernel,
    mesh=_SC_MESH,
    compiler_params=_SC_PARAMS,
    out_type=jax.ShapeDtypeStruct((E * N + 8, DIM), jnp.float32),
    scratch_types=[
        pltpu.VMEM((CHW,), jnp.float32),
        pltpu.VMEM((CHW,), jnp.int32),
        pltpu.VMEM((CHW,), jnp.int32),
        pltpu.VMEM((CHW,), jnp.int32),
        pltpu.VMEM((CHW, DIM), jnp.float32),
        pltpu.SemaphoreType.DMA,
        pltpu.SemaphoreType.DMA,
    ],
)
def _dispatch(wT_hbm, posT_hbm, einp_hbm, ginp_hbm,
              wv, pv, sidx, didx, stage, sem_g, sem_s):
    wid = lax.axis_index("s") * NC + lax.axis_index("c")
    base = wid * CHW
    lanes = lax.broadcasted_iota(jnp.int32, (L,), 0)
    for e in range(E):
        pltpu.sync_copy(wT_hbm.at[e, pl.ds(base, CHW)], wv)
        pltpu.sync_copy(posT_hbm.at[e, pl.ds(base, CHW)], pv)
        for g in range(NG):
            wvec = wv[pl.ds(g * L, L)]
            pvec = pv[pl.ds(g * L, L)]
            m = wvec > 0.0
            tok = base + g * L + lanes
            sidx[pl.ds(g * L, L)] = jnp.where(m, e * N + tok, 0)
            didx[pl.ds(g * L, L)] = jnp.where(m, e * N + pvec, E * N)
        pltpu.async_copy(einp_hbm.at[sidx], stage, sem_g).wait()
        pltpu.async_copy(stage, ginp_hbm.at[didx], sem_s).wait()


def _expert_kernel(cnt_ref, ginp_ref, w1_ref, b1_ref, w2_ref, b2_ref,
                   out_ref, acc_ref):
    e = pl.program_id(0)
    k = pl.program_id(1)
    j = pl.program_id(2)

    active = j * BTC < cnt_ref[e]

    @pl.when(active)
    def _work():
        xb = ginp_ref[...].astype(jnp.bfloat16)
        w1b = w1_ref[0].astype(jnp.bfloat16)
        h = lax.dot_general(xb, w1b, (((1,), (0,)), ((), ())),
                            preferred_element_type=jnp.float32)
        h = jax.nn.gelu(h + b1_ref[0, 0])
        hb = h.astype(jnp.bfloat16)
        w2b = w2_ref[0].astype(jnp.bfloat16)
        contrib = lax.dot_general(hb, w2b, (((1,), (0,)), ((), ())),
                                  preferred_element_type=jnp.float32)
        rows = pl.ds(j * BTC, BTC)
        prev = jnp.where(k == 0, 0.0, acc_ref[rows, :])
        total = prev + contrib
        acc_ref[rows, :] = total

        @pl.when(k == NK - 1)
        def _emit():
            out_ref[...] = total + b2_ref[pl.ds(e, 1), :]

    @pl.when(jnp.logical_not(active) & (j == 0) & (k == NK - 1))
    def _zero_block():
        out_ref[...] = jnp.zeros((BTC, DIM), jnp.float32)


def _experts(counts, ginp2d, W1, b1, W2, b2):
    def _jc(e, j, cnt):
        nblk = (cnt[e] + BTC - 1) // BTC
        return jnp.minimum(j, jnp.maximum(nblk - 1, 0))

    grid_spec = pltpu.PrefetchScalarGridSpec(
        num_scalar_prefetch=1,
        grid=(E, NK, NJ),
        in_specs=[
            pl.BlockSpec((BTC, DIM), lambda e, k, j, cnt: (e * NJ + _jc(e, j, cnt), 0)),
            pl.BlockSpec((1, DIM, FFB), lambda e, k, j, cnt: (e, 0, k)),
            pl.BlockSpec((1, 1, FFB), lambda e, k, j, cnt: (e, 0, k)),
            pl.BlockSpec((1, FFB, DIM), lambda e, k, j, cnt: (e, k, 0)),
            pl.BlockSpec((E, DIM), lambda e, k, j, cnt: (0, 0)),
        ],
        out_specs=pl.BlockSpec((BTC, DIM), lambda e, k, j, cnt: (e * NJ + _jc(e, j, cnt), 0)),
        scratch_shapes=[pltpu.VMEM((N, DIM), jnp.float32)],
    )
    return pl.pallas_call(
        _expert_kernel,
        grid_spec=grid_spec,
        out_shape=jax.ShapeDtypeStruct((E * N, DIM), jnp.float32),
    )(counts, ginp2d, W1, b1.reshape(E, 1, FF), W2, b2)


@functools.partial(
    pl.kernel,
    mesh=_SC_MESH,
    compiler_params=_SC_PARAMS,
    out_type=jax.ShapeDtypeStruct((N * DIM,), jnp.float32),
    scratch_types=[
        pltpu.VMEM((CHW * E,), jnp.float32),
        pltpu.VMEM((CHW * E,), jnp.int32),
        pltpu.VMEM((CHW * DIM,), jnp.float32),
        pltpu.VMEM((L, DIM), jnp.float32),
        pltpu.SemaphoreType.DMA,
    ],
)
def _combine(wf_hbm, pf_hbm, eout_hbm, fused_hbm, wv, pv, acc, stage, sem):
    wid = lax.axis_index("s") * NC + lax.axis_index("c")
    base = wid * CHW
    pltpu.sync_copy(wf_hbm.at[pl.ds(base * E, CHW * E)], wv)
    pltpu.sync_copy(pf_hbm.at[pl.ds(base * E, CHW * E)], pv)

    zv = jnp.zeros((L,), jnp.float32)
    for j in range(L):
        for v in range(DIM // L):
            stage[j, pl.ds(v * L, L)] = zv

    lanes = lax.broadcasted_iota(jnp.int32, (L,), 0)
    evec = lanes & (E - 1)

    def _pair(p, carry):
        wvec = wv[pl.ds(p * L, L)]
        pvec = pv[pl.ds(p * L, L)]
        m = wvec > 0.0
        gidx = evec * N + jnp.where(m, pvec, 0)
        pltpu.async_copy(eout_hbm.at[gidx], stage, sem).wait()
        splats = [
            plsc.load_gather(wv, [jnp.full((L,), p * L + j, jnp.int32)])
            for j in range(L)
        ]
        for v in range(DIM // L):
            sl = pl.ds(v * L, L)
            a0 = splats[0] * stage[0, sl]
            a1 = splats[E] * stage[E, sl]
            for j in range(1, E):
                a0 = a0 + splats[j] * stage[j, sl]
                a1 = a1 + splats[E + j] * stage[E + j, sl]
            acc[pl.ds((2 * p) * DIM + v * L, L)] = a0
            acc[pl.ds((2 * p + 1) * DIM + v * L, L)] = a1
        return carry

    lax.fori_loop(0, CHW // 2, _pair, 0)
    pltpu.sync_copy(acc, fused_hbm.at[pl.ds(base * DIM, CHW * DIM)])


def kernel(x, expert_inputs, Wg, bg, W1, b1, W2, b2):
    scoresT, wT, posT, counts = _gate(x, Wg, bg)

    einp2d = expert_inputs.reshape(E * N, DIM)
    ginp2d = _dispatch(wT, posT, einp2d)
    eout2d = _experts(counts.reshape(E), ginp2d, W1, b1, W2, b2)

    wflat = wT.T.reshape(N * E)
    pflat = posT.T.reshape(N * E)
    fused = _combine(wflat, pflat, eout2d)
    return (fused.reshape(N, DIM), scoresT.T)


# speedup vs baseline: 6.4805x; 6.4805x over previous
import jax
import jax.numpy as jnp
from jax import lax
from jax.experimental import pallas as pl
from jax.experimental.pallas import tpu as pltpu

N = 2048
DIM = 768
E = 8
FF = DIM * 4
THRESHOLD = 0.3

BTG = 256
NIG = N // BTG

BTC = 384
NJ = -(-N // BTC)
FFB = 1536
NK = FF // FFB


def _gate_kernel(x_ref, wg_ref, bg_ref, scores_ref, w_ref, pos_ref, cnt_ref,
                 carry_ref):
    i = pl.program_id(0)

    @pl.when(i == 0)
    def _init():
        carry_ref[...] = jnp.zeros((E, 1), jnp.float32)

    logits = lax.dot_general(wg_ref[...], x_ref[...],
                             (((0,), (1,)), ((), ())),
                             preferred_element_type=jnp.float32)
    logits = logits + bg_ref[...]
    mx0 = jnp.max(logits, axis=0, keepdims=True)
    ex = jnp.exp(logits - mx0)
    scores = ex / jnp.sum(ex, axis=0, keepdims=True)
    scores_ref[...] = scores

    mask = (scores >= THRESHOLD).astype(jnp.float32)
    masked = scores * mask
    denom_raw = jnp.sum(masked, axis=0, keepdims=True)
    normed = masked / (denom_raw + 1e-6)
    iot = lax.broadcasted_iota(jnp.int32, scores.shape, 0)
    mxs = jnp.max(scores, axis=0, keepdims=True)
    cand = jnp.where(scores == mxs, iot, E)
    top1 = jnp.min(cand, axis=0, keepdims=True)
    onehot = (iot == top1).astype(jnp.float32)
    w = jnp.where(denom_raw == 0.0, onehot, normed)
    w_ref[...] = w

    act = (w > 0.0).astype(jnp.float32)
    rowi = lax.broadcasted_iota(jnp.int32, (BTG, BTG), 0)
    coli = lax.broadcasted_iota(jnp.int32, (BTG, BTG), 1)
    tri = (rowi < coli).astype(jnp.float32)
    pos = lax.dot_general(act, tri, (((1,), (0,)), ((), ())),
                          preferred_element_type=jnp.float32)
    pos = pos + carry_ref[...]
    pos_ref[...] = pos.astype(jnp.int32)
    new_carry = carry_ref[...] + jnp.sum(act, axis=1, keepdims=True)
    carry_ref[...] = new_carry

    @pl.when(i == NIG - 1)
    def _fin():
        cnt_ref[...] = new_carry.astype(jnp.int32)


def _gate(x, Wg, bg):
    return pl.pallas_call(
        _gate_kernel,
        grid=(NIG,),
        in_specs=[
            pl.BlockSpec((BTG, DIM), lambda i: (i, 0)),
            pl.BlockSpec((DIM, E), lambda i: (0, 0)),
            pl.BlockSpec((E, 1), lambda i: (0, 0)),
        ],
        out_specs=(
            pl.BlockSpec((E, BTG), lambda i: (0, i)),
            pl.BlockSpec((E, BTG), lambda i: (0, i)),
            pl.BlockSpec((E, BTG), lambda i: (0, i)),
            pl.BlockSpec((E, 1), lambda i: (0, 0)),
        ),
        out_shape=(
            jax.ShapeDtypeStruct((E, N), jnp.float32),
            jax.ShapeDtypeStruct((E, N), jnp.float32),
            jax.ShapeDtypeStruct((E, N), jnp.int32),
            jax.ShapeDtypeStruct((E, 1), jnp.int32),
        ),
        scratch_shapes=[pltpu.VMEM((E, 1), jnp.float32)],
    )(x, Wg, bg.reshape(E, 1))


def _expert_kernel(cnt_ref, einp_ref, w1_ref, b1_ref, w2_ref, b2_ref,
                   wT_ref, posT_ref, out_ref, xg_ref, o_ref):
    e = pl.program_id(0)
    j = pl.program_id(1)
    k = pl.program_id(2)
    first = (e == 0) & (j == 0) & (k == 0)

    @pl.when(first)
    def _init():
        out_ref[...] = jnp.zeros((N, DIM), jnp.float32)

    active = j * BTC < cnt_ref[e]

    @pl.when(active)
    def _work():
        pos = posT_ref[pl.ds(e, 1), :]
        w = wT_ref[pl.ds(e, 1), :]
        act = w > 0.0
        rr = lax.broadcasted_iota(jnp.int32, (BTC, N), 0) + j * BTC
        onehot = (pos == rr) & act

        @pl.when(k == 0)
        def _gather():
            pmat = onehot.astype(jnp.float32)
            xg_ref[...] = lax.dot_general(
                pmat, einp_ref[0], (((1,), (0,)), ((), ())),
                preferred_element_type=jnp.float32)

        h = lax.dot_general(xg_ref[...], w1_ref[0], (((1,), (0,)), ((), ())),
                            preferred_element_type=jnp.float32)
        h = jax.nn.gelu(h + b1_ref[0, 0])
        contrib = lax.dot_general(h, w2_ref[0], (((1,), (0,)), ((), ())),
                                  preferred_element_type=jnp.float32)
        prev = jnp.where(k == 0, 0.0, o_ref[...])
        total = prev + contrib
        o_ref[...] = total

        @pl.when(k == NK - 1)
        def _fuse():
            gmat = jnp.where(onehot, w, 0.0)
            opb = total + b2_ref[pl.ds(e, 1), :]
            out_ref[...] += lax.dot_general(
                gmat, opb, (((0,), (0,)), ((), ())),
                preferred_element_type=jnp.float32)


def _experts(counts, expert_inputs, W1, b1, W2, b2, wT, posT):
    def _jc(e, j, cnt):
        nblk = lax.div(cnt[e] + BTC - 1, BTC)
        return jnp.minimum(j, jnp.maximum(nblk - 1, 0))

    def _kc(e, j, k, cnt):
        return jnp.where(j * BTC < cnt[e], k, NK - 1)

    grid_spec = pltpu.PrefetchScalarGridSpec(
        num_scalar_prefetch=1,
        grid=(E, NJ, NK),
        in_specs=[
            pl.BlockSpec((1, N, DIM), lambda e, j, k, cnt: (e, 0, 0)),
            pl.BlockSpec((1, DIM, FFB), lambda e, j, k, cnt: (e, 0, _kc(e, j, k, cnt))),
            pl.BlockSpec((1, 1, FFB), lambda e, j, k, cnt: (e, 0, _kc(e, j, k, cnt))),
            pl.BlockSpec((1, FFB, DIM), lambda e, j, k, cnt: (e, _kc(e, j, k, cnt), 0)),
            pl.BlockSpec((E, DIM), lambda e, j, k, cnt: (0, 0)),
            pl.BlockSpec((E, N), lambda e, j, k, cnt: (0, 0)),
            pl.BlockSpec((E, N), lambda e, j, k, cnt: (0, 0)),
        ],
        out_specs=pl.BlockSpec((N, DIM), lambda e, j, k, cnt: (0, 0)),
        scratch_shapes=[
            pltpu.VMEM((BTC, DIM), jnp.float32),
            pltpu.VMEM((BTC, DIM), jnp.float32),
        ],
    )
    return pl.pallas_call(
        _expert_kernel,
        grid_spec=grid_spec,
        out_shape=jax.ShapeDtypeStruct((N, DIM), jnp.float32),
    )(counts, expert_inputs, W1, b1.reshape(E, 1, FF), W2, b2, wT, posT)


def kernel(x, expert_inputs, Wg, bg, W1, b1, W2, b2):
    scoresT, wT, posT, counts = _gate(x, Wg, bg)
    fused = _experts(counts.reshape(E), expert_inputs, W1, b1, W2, b2,
                     wT, posT)
    return (fused, scoresT.T)


# speedup vs baseline: 8.4327x; 1.3012x over previous
import jax
import jax.numpy as jnp
from jax import lax
from jax.experimental import pallas as pl
from jax.experimental.pallas import tpu as pltpu

N = 2048
DIM = 768
E = 8
FF = DIM * 4
THRESHOLD = 0.3

BTG = 256
NIG = N // BTG

BTC = 384
NJ = -(-N // BTC)
FFB = 768
NK = FF // FFB


def _gate_kernel(x_ref, wg_ref, bg_ref, scores_ref, w_ref, pos_ref, cnt_ref,
                 carry_ref):
    i = pl.program_id(0)

    @pl.when(i == 0)
    def _init():
        carry_ref[...] = jnp.zeros((E, 1), jnp.float32)

    logits = lax.dot_general(wg_ref[...], x_ref[...],
                             (((0,), (1,)), ((), ())),
                             preferred_element_type=jnp.float32)
    logits = logits + bg_ref[...]
    mx0 = jnp.max(logits, axis=0, keepdims=True)
    ex = jnp.exp(logits - mx0)
    scores = ex / jnp.sum(ex, axis=0, keepdims=True)
    scores_ref[...] = scores

    mask = (scores >= THRESHOLD).astype(jnp.float32)
    masked = scores * mask
    denom_raw = jnp.sum(masked, axis=0, keepdims=True)
    normed = masked / (denom_raw + 1e-6)
    iot = lax.broadcasted_iota(jnp.int32, scores.shape, 0)
    mxs = jnp.max(scores, axis=0, keepdims=True)
    cand = jnp.where(scores == mxs, iot, E)
    top1 = jnp.min(cand, axis=0, keepdims=True)
    onehot = (iot == top1).astype(jnp.float32)
    w = jnp.where(denom_raw == 0.0, onehot, normed)
    w_ref[...] = w

    act = (w > 0.0).astype(jnp.float32)
    rowi = lax.broadcasted_iota(jnp.int32, (BTG, BTG), 0)
    coli = lax.broadcasted_iota(jnp.int32, (BTG, BTG), 1)
    tri = (rowi < coli).astype(jnp.float32)
    pos = lax.dot_general(act, tri, (((1,), (0,)), ((), ())),
                          preferred_element_type=jnp.float32)
    pos = pos + carry_ref[...]
    pos_ref[...] = pos.astype(jnp.int32)
    new_carry = carry_ref[...] + jnp.sum(act, axis=1, keepdims=True)
    carry_ref[...] = new_carry

    @pl.when(i == NIG - 1)
    def _fin():
        cnt_ref[...] = new_carry.astype(jnp.int32)


def _gate(x, Wg, bg):
    return pl.pallas_call(
        _gate_kernel,
        grid=(NIG,),
        in_specs=[
            pl.BlockSpec((BTG, DIM), lambda i: (i, 0)),
            pl.BlockSpec((DIM, E), lambda i: (0, 0)),
            pl.BlockSpec((E, 1), lambda i: (0, 0)),
        ],
        out_specs=(
            pl.BlockSpec((E, BTG), lambda i: (0, i)),
            pl.BlockSpec((E, BTG), lambda i: (0, i)),
            pl.BlockSpec((E, BTG), lambda i: (0, i)),
            pl.BlockSpec((E, 1), lambda i: (0, 0)),
        ),
        out_shape=(
            jax.ShapeDtypeStruct((E, N), jnp.float32),
            jax.ShapeDtypeStruct((E, N), jnp.float32),
            jax.ShapeDtypeStruct((E, N), jnp.int32),
            jax.ShapeDtypeStruct((E, 1), jnp.int32),
        ),
        scratch_shapes=[pltpu.VMEM((E, 1), jnp.float32)],
    )(x, Wg, bg.reshape(E, 1))


def _expert_kernel(cnt_ref, einp_ref, w1_ref, b1_ref, w2_ref, b2_ref,
                   wT_ref, posT_ref, out_ref, xg_ref, o_ref):
    e = pl.program_id(0)
    k = pl.program_id(1)

    @pl.when((e == 0) & (k == 0))
    def _init():
        out_ref[...] = jnp.zeros((N, DIM), jnp.float32)

    nblk = lax.div(cnt_ref[e] + BTC - 1, BTC)
    pos = posT_ref[pl.ds(e, 1), :]
    w = wT_ref[pl.ds(e, 1), :]
    act = w > 0.0

    @pl.when(k == 0)
    def _gather():
        def gbody(j, c):
            rr = lax.broadcasted_iota(jnp.int32, (BTC, N), 0) + j * BTC
            pmat = ((pos == rr) & act).astype(jnp.float32)
            xg_ref[pl.ds(j * BTC, BTC), :] = lax.dot_general(
                pmat, einp_ref[0], (((1,), (0,)), ((), ())),
                preferred_element_type=jnp.float32)
            return c

        lax.fori_loop(0, nblk, gbody, 0)

    def fbody(j, c):
        rows = pl.ds(j * BTC, BTC)
        h = lax.dot_general(xg_ref[rows, :], w1_ref[0],
                            (((1,), (0,)), ((), ())),
                            preferred_element_type=jnp.float32)
        h = jax.nn.gelu(h + b1_ref[0, 0])
        contrib = lax.dot_general(h, w2_ref[0], (((1,), (0,)), ((), ())),
                                  preferred_element_type=jnp.float32)
        prev = jnp.where(k == 0, 0.0, o_ref[rows, :])
        total = prev + contrib
        o_ref[rows, :] = total

        @pl.when(k == NK - 1)
        def _fuse():
            rr = lax.broadcasted_iota(jnp.int32, (BTC, N), 0) + j * BTC
            gmat = jnp.where((pos == rr) & act, w, 0.0)
            opb = total + b2_ref[pl.ds(e, 1), :]
            out_ref[...] += lax.dot_general(
                gmat, opb, (((0,), (0,)), ((), ())),
                preferred_element_type=jnp.float32)

        return c

    lax.fori_loop(0, nblk, fbody, 0)


def _experts(counts, expert_inputs, W1, b1, W2, b2, wT, posT):
    grid_spec = pltpu.PrefetchScalarGridSpec(
        num_scalar_prefetch=1,
        grid=(E, NK),
        in_specs=[
            pl.BlockSpec((1, N, DIM), lambda e, k, cnt: (e, 0, 0)),
            pl.BlockSpec((1, DIM, FFB), lambda e, k, cnt: (e, 0, k)),
            pl.BlockSpec((1, 1, FFB), lambda e, k, cnt: (e, 0, k)),
            pl.BlockSpec((1, FFB, DIM), lambda e, k, cnt: (e, k, 0)),
            pl.BlockSpec((E, DIM), lambda e, k, cnt: (0, 0)),
            pl.BlockSpec((E, N), lambda e, k, cnt: (0, 0)),
            pl.BlockSpec((E, N), lambda e, k, cnt: (0, 0)),
        ],
        out_specs=pl.BlockSpec((N, DIM), lambda e, k, cnt: (0, 0)),
        scratch_shapes=[
            pltpu.VMEM((NJ * BTC, DIM), jnp.float32),
            pltpu.VMEM((NJ * BTC, DIM), jnp.float32),
        ],
    )
    return pl.pallas_call(
        _expert_kernel,
        grid_spec=grid_spec,
        out_shape=jax.ShapeDtypeStruct((N, DIM), jnp.float32),
    )(counts, expert_inputs, W1, b1.reshape(E, 1, FF), W2, b2, wT, posT)


def kernel(x, expert_inputs, Wg, bg, W1, b1, W2, b2):
    scoresT, wT, posT, counts = _gate(x, Wg, bg)
    fused = _experts(counts.reshape(E), expert_inputs, W1, b1, W2, b2,
                     wT, posT)
    return (fused, scoresT.T)
